# Initial kernel scaffold; baseline (speedup 1.0000x reference)
import jax
import jax.numpy as jnp
from jax.experimental import pallas as pl


def kernel(z, W, emb):
    raise NotImplementedError("write your pallas kernel here")


# speedup vs baseline: 4.0295x; 4.0295x over previous
import functools

import jax
import jax.numpy as jnp
from jax import lax
from jax.experimental import pallas as pl
from jax.experimental.pallas import tpu as pltpu
from jax.experimental.pallas import tpu_sc as plsc

B, C_IN, N = 2, 192, 1024
D, K = 64, 512

_NC, _NS = 2, 16
_NW = _NC * _NS
_TOK = B * N
_TPW = _TOK // _NW


def _tc_body(z_ref, w_ref, emb_ref, idx_ref):
    zb = z_ref[0]
    w = w_ref[...]
    emb = emb_ref[...]
    hi = lax.Precision.HIGHEST
    ze = jnp.dot(w.astype(jnp.bfloat16), zb.astype(jnp.bfloat16),
                 preferred_element_type=jnp.float32)
    scores = jnp.dot(emb, ze, preferred_element_type=jnp.float32,
                     precision=hi)
    esq = jnp.sum(emb * emb, axis=1, keepdims=True)
    dist = esq - 2.0 * scores

    iota = lax.broadcasted_iota(jnp.int32, (K, N), 0)
    m1 = jnp.min(dist, axis=0, keepdims=True)
    i1 = jnp.min(jnp.where(dist == m1, iota, K), axis=0, keepdims=True)
    dist2 = jnp.where(iota == i1, jnp.float32(jnp.inf), dist)
    m2 = jnp.min(dist2, axis=0, keepdims=True)
    i2 = jnp.min(jnp.where(dist2 == m2, iota, K), axis=0, keepdims=True)

    oh1 = (iota == i1).astype(jnp.float32)
    oh2 = (iota == i2).astype(jnp.float32)
    dn = (((0,), (0,)), ((), ()))
    e1 = lax.dot_general(emb, oh1, dn, precision=hi,
                         preferred_element_type=jnp.float32)
    e2 = lax.dot_general(emb, oh2, dn, precision=hi,
                         preferred_element_type=jnp.float32)
    d1 = jnp.sum((ze - e1) ** 2, axis=0, keepdims=True)
    d2 = jnp.sum((ze - e2) ** 2, axis=0, keepdims=True)
    pick2 = (d2 < d1) | ((d2 == d1) & (i2 < i1))
    idx_ref[0] = jnp.where(pick2, i2, i1)


_tc_call = pl.pallas_call(
    _tc_body,
    grid=(B,),
    in_specs=[
        pl.BlockSpec((1, C_IN, N), lambda b: (b, 0, 0)),
        pl.BlockSpec((D, C_IN), lambda b: (0, 0)),
        pl.BlockSpec((K, D), lambda b: (0, 0)),
    ],
    out_specs=pl.BlockSpec((1, 1, N), lambda b: (b, 0, 0)),
    out_shape=jax.ShapeDtypeStruct((B, 1, N), jnp.int32),
)


_DP = 128


@functools.cache
def _make_sc_gather():
    mesh = plsc.VectorSubcoreMesh(core_axis_name="c", subcore_axis_name="s")

    @functools.partial(
        pl.kernel,
        mesh=mesh,
        out_type=jax.ShapeDtypeStruct((_TOK, _DP), jnp.float32),
        scratch_types=[
            pltpu.VMEM((_TPW,), jnp.int32),
            pltpu.VMEM((_TPW, _DP), jnp.float32),
            pltpu.SemaphoreType.DMA,
        ],
    )
    def _sc_gather(emb_hbm, idx_hbm, out_hbm, idx_v, rows_v, sem):
        wid = lax.axis_index("s") * _NC + lax.axis_index("c")
        base = wid * _TPW
        pltpu.sync_copy(idx_hbm.at[pl.ds(base, _TPW)], idx_v)
        pltpu.async_copy(emb_hbm.at[idx_v], rows_v, sem).wait()
        pltpu.sync_copy(rows_v, out_hbm.at[pl.ds(base, _TPW)])

    return _sc_gather


def kernel(z, W, emb):
    idx = _tc_call(z, W, emb).reshape(_TOK)
    emb_p = jnp.pad(emb, ((0, 0), (0, _DP - D)))
    rows = _make_sc_gather()(emb_p, idx)
    return rows[:, :D].reshape(B, N, D).transpose(0, 2, 1)
